# baseline (speedup 1.0000x reference)
import functools

import jax
import jax.numpy as jnp
from jax import lax
from jax.experimental import pallas as pl
from jax.experimental.pallas import tpu as pltpu
from jax.experimental.pallas import tpu_sc as plsc


@functools.partial(jax.jit, static_argnames=("B", "E", "D", "NC", "NS"))
def _routing_gather(table, weights, *, B, E, D, NC, NS):
    NW = NC * NS
    b_per_w = B // NW
    K = 32
    n_groups = b_per_w // K
    L = 16

    mesh = plsc.VectorSubcoreMesh(core_axis_name="c", subcore_axis_name="s")

    @functools.partial(
        pl.kernel,
        out_type=jax.ShapeDtypeStruct((B, D), jnp.float32),
        mesh=mesh,
        compiler_params=pltpu.CompilerParams(needs_layout_passes=False),
        scratch_types=[
            pltpu.VMEM((E, D), jnp.float32),
            pltpu.VMEM((b_per_w, E), jnp.float32),
            pltpu.VMEM((K, D), jnp.float32),
            pltpu.SemaphoreType.DMA,
            pltpu.SemaphoreType.DMA,
        ],
    )
    def k(table_hbm, w_hbm, out_hbm, table_v, w_v, dummy_v, sem_t, sem_w):
        wid = lax.axis_index("s") * NC + lax.axis_index("c")
        base = wid * b_per_w

        tcopy = pltpu.async_copy(table_hbm, table_v, sem_t)
        pltpu.sync_copy(w_hbm.at[pl.ds(base, b_per_w), :], w_v)
        tcopy.wait()

        lane = lax.iota(jnp.int32, L)

        def drain_group():
            pltpu.make_async_copy(
                out_hbm.at[pl.ds(0, K), :], dummy_v, sem_w).wait()

        nib = (lane & 7) * 4

        @pl.loop(0, n_groups)
        def _group(g):
            for h in range(K // L):
                rows = g * K + h * L + lane
                best_v = plsc.load_gather(
                    w_v, [rows, jnp.zeros((L,), jnp.int32)])
                best_e = jnp.zeros((L,), jnp.int32)
                for e in range(1, E):
                    v = plsc.load_gather(
                        w_v, [rows, jnp.full((L,), e, jnp.int32)])
                    better = v > best_v
                    best_v = jnp.where(better, v, best_v)
                    best_e = jnp.where(
                        better, jnp.full((L,), e, jnp.int32), best_e)

                sh = best_e << nib
                w0 = jnp.sum(jnp.where(lane < 8, sh, 0), axis=0)
                w1 = jnp.sum(jnp.where(lane >= 8, sh, 0), axis=0)
                for j in range(L):
                    e = ((w0 if j < 8 else w1) >> ((j & 7) * 4)) & 7
                    pltpu.async_copy(
                        table_v.at[e],
                        out_hbm.at[base + g * K + h * L + j], sem_w)

            @pl.when(g > 1)
            def _():
                drain_group()

        drain_group()
        drain_group()

    return k(table, weights)


def kernel(opinions, weights):
    E, B, D = opinions.shape
    info = plsc.get_sparse_core_info()
    table = opinions[0, :E, :]
    return _routing_gather(
        table, weights, B=B, E=E, D=D,
        NC=info.num_cores, NS=info.num_subcores)


# speedup vs baseline: 1.0064x; 1.0064x over previous
import functools

import jax
import jax.numpy as jnp
from jax import lax
from jax.experimental import pallas as pl
from jax.experimental.pallas import tpu as pltpu
from jax.experimental.pallas import tpu_sc as plsc


@functools.partial(jax.jit, static_argnames=("B", "E", "D", "NC", "NS"))
def _routing_gather(table, weights, *, B, E, D, NC, NS):
    NW = NC * NS
    b_per_w = B // NW
    K = 32
    n_groups = b_per_w // K
    L = 16

    mesh = plsc.VectorSubcoreMesh(core_axis_name="c", subcore_axis_name="s")

    @functools.partial(
        pl.kernel,
        out_type=jax.ShapeDtypeStruct((B, D), jnp.float32),
        mesh=mesh,
        compiler_params=pltpu.CompilerParams(
            needs_layout_passes=False,
            disable_bounds_checks=True,
            disable_semaphore_checks=True,
        ),
        scratch_types=[
            pltpu.VMEM((E, D), jnp.float32),
            pltpu.VMEM((b_per_w, E), jnp.float32),
            pltpu.VMEM((K, D), jnp.float32),
            pltpu.SemaphoreType.DMA,
            pltpu.SemaphoreType.DMA,
        ],
    )
    def k(table_hbm, w_hbm, out_hbm, table_v, w_v, dummy_v, sem_t, sem_w):
        wid = lax.axis_index("s") * NC + lax.axis_index("c")
        base = wid * b_per_w

        tcopy = pltpu.async_copy(table_hbm, table_v, sem_t)
        pltpu.sync_copy(w_hbm.at[pl.ds(base, b_per_w), :], w_v)

        lane = lax.iota(jnp.int32, L)

        def drain_group():
            pltpu.make_async_copy(
                out_hbm.at[pl.ds(0, K), :], dummy_v, sem_w).wait()

        nib = (lane & 7) * 4

        @pl.loop(0, n_groups)
        def _group(g):
            for h in range(K // L):
                rows = g * K + h * L + lane
                best_v = plsc.load_gather(
                    w_v, [rows, jnp.zeros((L,), jnp.int32)])
                best_e = jnp.zeros((L,), jnp.int32)
                for e in range(1, E):
                    v = plsc.load_gather(
                        w_v, [rows, jnp.full((L,), e, jnp.int32)])
                    better = v > best_v
                    best_v = jnp.where(better, v, best_v)
                    best_e = jnp.where(
                        better, jnp.full((L,), e, jnp.int32), best_e)

                sh = best_e << nib
                w0 = jnp.sum(jnp.where(lane < 8, sh, 0), axis=0)
                w1 = jnp.sum(jnp.where(lane >= 8, sh, 0), axis=0)
                if h == 0:
                    @pl.when(g == 0)
                    def _():
                        tcopy.wait()
                for j in range(L):
                    e = ((w0 if j < 8 else w1) >> ((j & 7) * 4)) & 7
                    pltpu.async_copy(
                        table_v.at[e],
                        out_hbm.at[base + g * K + h * L + j], sem_w)

            @pl.when(g > 1)
            def _():
                drain_group()

        drain_group()
        drain_group()

    return k(table, weights)


def kernel(opinions, weights):
    E, B, D = opinions.shape
    info = plsc.get_sparse_core_info()
    table = opinions[0, :E, :]
    return _routing_gather(
        table, weights, B=B, E=E, D=D,
        NC=info.num_cores, NS=info.num_subcores)
